# Initial kernel scaffold; baseline (speedup 1.0000x reference)
import jax
import jax.numpy as jnp
from jax.experimental import pallas as pl


def kernel(x, w1, b1, w2, b2, w3, b3, w4, b4, w5, b5):
    raise NotImplementedError("write your pallas kernel here")


# speedup vs baseline: 22.6157x; 22.6157x over previous
import functools

import jax
import jax.numpy as jnp
from jax.experimental import pallas as pl
from jax.experimental.pallas import tpu as pltpu


def _s2d_pad(h):
    B, H, W, C = h.shape
    xp = jnp.pad(h, ((0, 0), (2, 2), (2, 2), (0, 0)))
    Hs, Ws = (H + 4) // 2, (W + 4) // 2
    xs = xp.reshape(B, Hs, 2, Ws, 2, C).transpose(0, 1, 3, 2, 4, 5)
    return xs.reshape(B, Hs, Ws, 4 * C).astype(jnp.bfloat16)


def _w_s2d(w):
    C, O = w.shape[2], w.shape[3]
    wp = jnp.pad(w, ((0, 1), (0, 1), (0, 0), (0, 0)))
    wt = wp.reshape(3, 2, 3, 2, C, O).transpose(0, 2, 1, 3, 4, 5)
    return wt.reshape(3, 3, 4 * C, O)


def _tap_pieces(C):
    pieces = []
    for a in range(3):
        for b in range(3):
            if a < 2 and b < 2:
                pieces.append((a, b, 0, 4 * C))
            elif a == 2 and b < 2:
                pieces.append((a, b, 0, 2 * C))
            elif a < 2 and b == 2:
                pieces.append((a, b, 0, C))
                pieces.append((a, b, 2 * C, 3 * C))
            else:
                pieces.append((a, b, 0, C))
    return pieces


def _pack_weight(w):
    C = w.shape[2]
    wt = _w_s2d(w)
    parts = [wt[a, b, c0:c1, :] for (a, b, c0, c1) in _tap_pieces(C)]
    return jnp.concatenate(parts, axis=0).astype(jnp.bfloat16)


def _gemm_bias_relu_kernel(p_ref, w_ref, b_ref, o_ref):
    acc = jnp.dot(p_ref[...], w_ref[...], preferred_element_type=jnp.float32)
    o_ref[...] = jnp.maximum(acc + b_ref[...], 0.0).astype(o_ref.dtype)


def _conv_s2d_kernel(Ho, Wo, C, x_ref, w_ref, b_ref, o_ref):
    x = x_ref[...]
    k = x.shape[0]
    parts = [
        x[:, a:a + Ho, b:b + Wo, c0:c1].reshape(k * Ho * Wo, c1 - c0)
        for (a, b, c0, c1) in _tap_pieces(C)
    ]
    p = jnp.concatenate(parts, axis=-1)
    acc = jnp.dot(p, w_ref[...], preferred_element_type=jnp.float32)
    y = jnp.maximum(acc + b_ref[...], 0.0)
    o_ref[...] = y.reshape(k, Ho, Wo, -1).astype(o_ref.dtype)


def _conv_head_kernel(Ho, Wo, C, x_ref, w_ref, b_ref, w5_ref, b5_ref, o_ref):
    x = x_ref[...]
    k = x.shape[0]
    parts = [
        x[:, a:a + Ho, b:b + Wo, c0:c1].reshape(k * Ho * Wo, c1 - c0)
        for (a, b, c0, c1) in _tap_pieces(C)
    ]
    p = jnp.concatenate(parts, axis=-1)
    acc = jnp.dot(p, w_ref[...], preferred_element_type=jnp.float32)
    h = jnp.maximum(acc + b_ref[...], 0.0)
    hb = h.reshape(k, Ho * Wo, -1)
    logits = jnp.sum(hb * w5_ref[...][None], axis=(1, 2)) + b5_ref[0, 0]
    o_ref[...] = jax.nn.sigmoid(logits).reshape(1, k, 1)


def _params(vmem_mb):
    return pltpu.CompilerParams(
        dimension_semantics=("parallel",),
        vmem_limit_bytes=vmem_mb << 20,
    )


def _conv1(patches, w_mat, bias, TM):
    M, K = patches.shape
    N = w_mat.shape[1]
    grid = M // TM
    return pl.pallas_call(
        _gemm_bias_relu_kernel,
        out_shape=jax.ShapeDtypeStruct((M, N), jnp.bfloat16),
        grid=(grid,),
        in_specs=[
            pl.BlockSpec((TM, K), lambda i: (i, 0)),
            pl.BlockSpec((K, N), lambda i: (0, 0)),
            pl.BlockSpec((1, N), lambda i: (0, 0)),
        ],
        out_specs=pl.BlockSpec((TM, N), lambda i: (i, 0)),
        compiler_params=_params(32),
    )(patches, w_mat, bias)


def _conv_s2d(xs, w_mat, bias, k):
    B, Hs, Ws, C4 = xs.shape
    C = C4 // 4
    Ho, Wo = Hs - 2, Ws - 2
    K, N = w_mat.shape
    return pl.pallas_call(
        functools.partial(_conv_s2d_kernel, Ho, Wo, C),
        out_shape=jax.ShapeDtypeStruct((B, Ho, Wo, N), jnp.bfloat16),
        grid=(B // k,),
        in_specs=[
            pl.BlockSpec((k, Hs, Ws, C4), lambda i: (i, 0, 0, 0)),
            pl.BlockSpec((K, N), lambda i: (0, 0)),
            pl.BlockSpec((1, N), lambda i: (0, 0)),
        ],
        out_specs=pl.BlockSpec((k, Ho, Wo, N), lambda i: (i, 0, 0, 0)),
        compiler_params=_params(40),
    )(xs, w_mat, bias)


def _conv_head(xs, w_mat, bias, w5_mat, b5, k):
    B, Hs, Ws, C4 = xs.shape
    C = C4 // 4
    Ho, Wo = Hs - 2, Ws - 2
    K, N = w_mat.shape
    out = pl.pallas_call(
        functools.partial(_conv_head_kernel, Ho, Wo, C),
        out_shape=jax.ShapeDtypeStruct((B // k, k, 1), jnp.float32),
        grid=(B // k,),
        in_specs=[
            pl.BlockSpec((k, Hs, Ws, C4), lambda i: (i, 0, 0, 0)),
            pl.BlockSpec((K, N), lambda i: (0, 0)),
            pl.BlockSpec((1, N), lambda i: (0, 0)),
            pl.BlockSpec(w5_mat.shape, lambda i: (0, 0)),
            pl.BlockSpec((1, 1), lambda i: (0, 0)),
        ],
        out_specs=pl.BlockSpec((1, k, 1), lambda i: (i, 0, 0)),
        compiler_params=_params(44),
    )(xs, w_mat, bias, w5_mat, b5)
    return out.reshape(B, 1)


def kernel(x, w1, b1, w2, b2, w3, b3, w4, b4, w5, b5):
    B = x.shape[0]
    H = x.shape[2]
    Ho = H // 2

    xh = jnp.transpose(x, (0, 2, 3, 1))
    xp = jnp.pad(xh, ((0, 0), (2, 2), (2, 2), (0, 0))).astype(jnp.bfloat16)
    taps = []
    for kh in range(5):
        for kw in range(5):
            taps.append(xp[:, kh:kh + 2 * Ho:2, kw:kw + 2 * Ho:2, :])
    p1 = jnp.concatenate(taps, axis=-1).reshape(B * Ho * Ho, 75)
    w1m = w1.reshape(75, -1).astype(jnp.bfloat16)
    a1 = _conv1(p1, w1m, b1, TM=16384)
    h = a1.reshape(B, Ho, Ho, -1)

    h = _conv_s2d(_s2d_pad(h), _pack_weight(w2), b2, k=4)
    h = _conv_s2d(_s2d_pad(h), _pack_weight(w3), b3, k=4)

    w5m = w5[:, 0].reshape(64, -1)
    return _conv_head(_s2d_pad(h), _pack_weight(w4), b4, w5m, b5, k=8)


# speedup vs baseline: 55.0805x; 2.4355x over previous
import functools

import jax
import jax.numpy as jnp
from jax.experimental import pallas as pl
from jax.experimental.pallas import tpu as pltpu


def _s2d_pad(h):
    B, H, W, C = h.shape
    xp = jnp.pad(h, ((0, 0), (2, 2), (2, 2), (0, 0)))
    Hs, Ws = (H + 4) // 2, (W + 4) // 2
    xs = xp.reshape(B, Hs, 2, Ws, 2, C).transpose(0, 1, 3, 2, 4, 5)
    return xs.reshape(B, Hs, Ws, 4 * C).astype(jnp.bfloat16)


def _w_s2d(w):
    C, O = w.shape[2], w.shape[3]
    wp = jnp.pad(w, ((0, 1), (0, 1), (0, 0), (0, 0)))
    wt = wp.reshape(3, 2, 3, 2, C, O).transpose(0, 2, 1, 3, 4, 5)
    return wt.reshape(3, 3, 4 * C, O)


def _tap_pieces(C):
    pieces = []
    for a in range(3):
        for b in range(3):
            if a < 2 and b < 2:
                pieces.append((a, b, 0, 4 * C))
            elif a == 2 and b < 2:
                pieces.append((a, b, 0, 2 * C))
            elif a < 2 and b == 2:
                pieces.append((a, b, 0, C))
                pieces.append((a, b, 2 * C, 3 * C))
            else:
                pieces.append((a, b, 0, C))
    return pieces


def _pack_weight(w):
    C = w.shape[2]
    wt = _w_s2d(w)
    parts = [wt[a, b, c0:c1, :] for (a, b, c0, c1) in _tap_pieces(C)]
    return jnp.concatenate(parts, axis=0).astype(jnp.bfloat16)


def _gemm_bias_relu_kernel(p_ref, w_ref, b_ref, o_ref):
    acc = jnp.dot(p_ref[...], w_ref[...], preferred_element_type=jnp.float32)
    o_ref[...] = jnp.maximum(acc + b_ref[...], 0.0).astype(o_ref.dtype)


def _conv_s2d_kernel(Ho, Wo, C, x_ref, w_ref, b_ref, o_ref):
    x = x_ref[...]
    k = x.shape[0]
    parts = [
        x[:, a:a + Ho, b:b + Wo, c0:c1].reshape(k * Ho * Wo, c1 - c0)
        for (a, b, c0, c1) in _tap_pieces(C)
    ]
    p = jnp.concatenate(parts, axis=-1)
    acc = jnp.dot(p, w_ref[...], preferred_element_type=jnp.float32)
    y = jnp.maximum(acc + b_ref[...], 0.0)
    o_ref[...] = y.reshape(k, Ho, Wo, -1).astype(o_ref.dtype)


def _conv_head_kernel(Ho, Wo, C, x_ref, w_ref, b_ref, w5_ref, b5_ref, o_ref):
    x = x_ref[...]
    k = x.shape[0]
    parts = [
        x[:, a:a + Ho, b:b + Wo, c0:c1].reshape(k * Ho * Wo, c1 - c0)
        for (a, b, c0, c1) in _tap_pieces(C)
    ]
    p = jnp.concatenate(parts, axis=-1)
    acc = jnp.dot(p, w_ref[...], preferred_element_type=jnp.float32)
    h = jnp.maximum(acc + b_ref[...], 0.0)
    hb = h.reshape(k, Ho * Wo, -1)
    logits = jnp.sum(hb * w5_ref[...][None], axis=(1, 2)) + b5_ref[0, 0]
    o_ref[...] = jax.nn.sigmoid(logits).reshape(1, k, 1)


def _params(vmem_mb):
    return pltpu.CompilerParams(
        dimension_semantics=("parallel",),
        vmem_limit_bytes=vmem_mb << 20,
    )


def _conv1(patches, w_mat, bias, TM):
    M, K = patches.shape
    N = w_mat.shape[1]
    grid = M // TM
    return pl.pallas_call(
        _gemm_bias_relu_kernel,
        out_shape=jax.ShapeDtypeStruct((M, N), jnp.bfloat16),
        grid=(grid,),
        in_specs=[
            pl.BlockSpec((TM, K), lambda i: (i, 0)),
            pl.BlockSpec((K, N), lambda i: (0, 0)),
            pl.BlockSpec((1, N), lambda i: (0, 0)),
        ],
        out_specs=pl.BlockSpec((TM, N), lambda i: (i, 0)),
        compiler_params=_params(32),
    )(patches, w_mat, bias)


def _conv_s2d(xs, w_mat, bias, k):
    B, Hs, Ws, C4 = xs.shape
    C = C4 // 4
    Ho, Wo = Hs - 2, Ws - 2
    K, N = w_mat.shape
    return pl.pallas_call(
        functools.partial(_conv_s2d_kernel, Ho, Wo, C),
        out_shape=jax.ShapeDtypeStruct((B, Ho, Wo, N), jnp.bfloat16),
        grid=(B // k,),
        in_specs=[
            pl.BlockSpec((k, Hs, Ws, C4), lambda i: (i, 0, 0, 0)),
            pl.BlockSpec((K, N), lambda i: (0, 0)),
            pl.BlockSpec((1, N), lambda i: (0, 0)),
        ],
        out_specs=pl.BlockSpec((k, Ho, Wo, N), lambda i: (i, 0, 0, 0)),
        compiler_params=_params(40),
    )(xs, w_mat, bias)


def _conv_head(xs, w_mat, bias, w5_mat, b5, k):
    B, Hs, Ws, C4 = xs.shape
    C = C4 // 4
    Ho, Wo = Hs - 2, Ws - 2
    K, N = w_mat.shape
    out = pl.pallas_call(
        functools.partial(_conv_head_kernel, Ho, Wo, C),
        out_shape=jax.ShapeDtypeStruct((B // k, k, 1), jnp.float32),
        grid=(B // k,),
        in_specs=[
            pl.BlockSpec((k, Hs, Ws, C4), lambda i: (i, 0, 0, 0)),
            pl.BlockSpec((K, N), lambda i: (0, 0)),
            pl.BlockSpec((1, N), lambda i: (0, 0)),
            pl.BlockSpec(w5_mat.shape, lambda i: (0, 0)),
            pl.BlockSpec((1, 1), lambda i: (0, 0)),
        ],
        out_specs=pl.BlockSpec((1, k, 1), lambda i: (i, 0, 0)),
        compiler_params=_params(44),
    )(xs, w_mat, bias, w5_mat, b5)
    return out.reshape(B, 1)


def kernel(x, w1, b1, w2, b2, w3, b3, w4, b4, w5, b5):
    B = x.shape[0]
    H = x.shape[2]
    Ho = H // 2

    xh = jnp.transpose(x, (0, 2, 3, 1))
    xf = jnp.pad(xh, ((0, 0), (2, 2), (2, 6), (0, 0))).astype(jnp.bfloat16)
    xfv = xf.reshape(B, H + 4, -1)
    wins = []
    for kh in range(5):
        r = xfv[:, kh:kh + 2 * Ho:2, :]
        w16 = [r[:, :, u * 24:u * 24 + 48] for u in range(Ho // 4)]
        wins.append(jnp.stack(w16, axis=2))
    p1 = jnp.concatenate(wins, axis=-1)
    p1 = p1.reshape(B * Ho * (Ho // 4), 240)

    n1 = w1.shape[3]
    w1q = jnp.zeros((5, 2, 8, 3, 4, n1), jnp.float32)
    for s in range(4):
        for kw in range(5):
            g, wp = divmod(2 * s + kw, 8)
            w1q = w1q.at[:, g, wp, :, s, :].set(w1[:, kw, :, :])
    w1m = w1q.reshape(240, 4 * n1).astype(jnp.bfloat16)
    b1t = jnp.tile(b1, (1, 4))
    a1 = _conv1(p1, w1m, b1t, TM=4096)
    h = a1.reshape(B, Ho, Ho // 4, 4, n1).reshape(B, Ho, Ho, n1)

    h = _conv_s2d(_s2d_pad(h), _pack_weight(w2), b2, k=4)
    h = _conv_s2d(_s2d_pad(h), _pack_weight(w3), b3, k=4)

    w5m = w5[:, 0].reshape(64, -1)
    return _conv_head(_s2d_pad(h), _pack_weight(w4), b4, w5m, b5, k=8)
